# Initial kernel scaffold; baseline (speedup 1.0000x reference)
import jax
import jax.numpy as jnp
from jax.experimental import pallas as pl


def kernel(h, edge_index, norm, W, b, W_res, b_res):
    raise NotImplementedError("write your pallas kernel here")


# speedup vs baseline: 7.1516x; 7.1516x over previous
import functools

import jax
import jax.numpy as jnp
from jax import lax
from jax.experimental import pallas as pl
from jax.experimental.pallas import tpu as pltpu
from jax.experimental.pallas import tpu_sc as plsc

N = 10000
E = 320000
D = 128

NC = 2
NS = 16
NW = NC * NS
EPW = E // NW
CH = 80
NCHUNK = EPW // CH
NPAD = 10240
RPT = NPAD // NS

MBLK = 1000


def _mm_body(h_ref, w_ref, wt_ref, norm_ref, hw_ref, res_ref):
    h = h_ref[...]
    hw_ref[...] = jnp.dot(h, w_ref[...], preferred_element_type=jnp.float32) * norm_ref[...]
    res_ref[...] = jnp.dot(h, wt_ref[...], preferred_element_type=jnp.float32)


_mm_call = pl.pallas_call(
    _mm_body,
    grid=(N // MBLK,),
    in_specs=[
        pl.BlockSpec((MBLK, D), lambda i: (i, 0)),
        pl.BlockSpec((D, D), lambda i: (0, 0)),
        pl.BlockSpec((D, D), lambda i: (0, 0)),
        pl.BlockSpec((MBLK, 1), lambda i: (i, 0)),
    ],
    out_specs=[
        pl.BlockSpec((MBLK, D), lambda i: (i, 0)),
        pl.BlockSpec((MBLK, D), lambda i: (i, 0)),
    ],
    out_shape=[
        jax.ShapeDtypeStruct((N, D), jnp.float32),
        jax.ShapeDtypeStruct((N, D), jnp.float32),
    ],
)


_sc_mesh = plsc.VectorSubcoreMesh(
    core_axis_name="c", subcore_axis_name="s", num_cores=NC, num_subcores=NS
)


@functools.partial(
    pl.kernel,
    out_type=jax.ShapeDtypeStruct((NC, NPAD, D), jnp.float32),
    mesh=_sc_mesh,
    scratch_types=[
        pltpu.VMEM((NCHUNK, CH), jnp.int32),
        pltpu.VMEM((NCHUNK, CH), jnp.int32),
        pltpu.VMEM((CH, D), jnp.float32),
        pltpu.VMEM_SHARED((NPAD, D), jnp.float32),
        pltpu.SemaphoreType.DMA,
    ],
)
def _sc_scatter(hw_hbm, src_hbm, dst_hbm, zeros_hbm, out_hbm,
                src_v, dst_v, rows_v, acc_sh, sem):
    cid = lax.axis_index("c")
    sid = lax.axis_index("s")
    wid = sid * NC + cid
    pltpu.sync_copy(zeros_hbm.at[pl.ds(sid * RPT, RPT)],
                    acc_sh.at[pl.ds(sid * RPT, RPT)])
    pltpu.sync_copy(src_hbm.at[wid], src_v)
    pltpu.sync_copy(dst_hbm.at[wid], dst_v)
    plsc.subcore_barrier()

    def body(j, carry):
        pltpu.async_copy(hw_hbm.at[src_v.at[j]], rows_v, sem).wait()
        pltpu.sync_copy(rows_v, acc_sh.at[dst_v.at[j]], add=True)
        return carry

    lax.fori_loop(0, NCHUNK, body, 0)
    plsc.subcore_barrier()
    pltpu.sync_copy(acc_sh.at[pl.ds(sid * RPT, RPT)],
                    out_hbm.at[cid, pl.ds(sid * RPT, RPT)])


def _fin_body(agg_ref, norm_ref, b_ref, res_ref, out_ref):
    agg = agg_ref[0] + agg_ref[1]
    out_ref[...] = jnp.maximum(agg * norm_ref[...] + b_ref[...] + res_ref[...], 0.0)


_fin_call = pl.pallas_call(
    _fin_body,
    grid=(N // MBLK,),
    in_specs=[
        pl.BlockSpec((NC, MBLK, D), lambda i: (0, i, 0)),
        pl.BlockSpec((MBLK, 1), lambda i: (i, 0)),
        pl.BlockSpec((1, D), lambda i: (0, 0)),
        pl.BlockSpec((MBLK, D), lambda i: (i, 0)),
    ],
    out_specs=pl.BlockSpec((MBLK, D), lambda i: (i, 0)),
    out_shape=jax.ShapeDtypeStruct((N, D), jnp.float32),
)


def kernel(h, edge_index, norm, W, b, W_res, b_res):
    src = edge_index[0].reshape(NW, NCHUNK, CH)
    dst = edge_index[1].reshape(NW, NCHUNK, CH)
    hw, res = _mm_call(h, W, W_res.T, norm)
    zeros = jnp.zeros((NPAD, D), jnp.float32)
    aggs = _sc_scatter(hw, src, dst, zeros)
    bias = (b + b_res).reshape(1, D)
    return _fin_call(aggs, norm, bias, res)


# speedup vs baseline: 10.6508x; 1.4893x over previous
import functools

import jax
import jax.numpy as jnp
from jax import lax
from jax.experimental import pallas as pl
from jax.experimental.pallas import tpu as pltpu
from jax.experimental.pallas import tpu_sc as plsc

N = 10000
E = 320000
D = 128

NC = 2
NS = 16
NW = NC * NS
EPW = E // NW
CH = 125
NCHUNK = EPW // CH
G = 16
NGROUP = NCHUNK // G
RSTRIDE = 624
RWIN = 640

MBLK = 1000


def _mm_body(h_ref, w_ref, wt_ref, norm_ref, hw_ref, res_ref):
    h = h_ref[...]
    hw_ref[...] = jnp.dot(h, w_ref[...], preferred_element_type=jnp.float32) * norm_ref[...]
    res_ref[...] = jnp.dot(h, wt_ref[...], preferred_element_type=jnp.float32)


_mm_call = pl.pallas_call(
    _mm_body,
    grid=(N // MBLK,),
    in_specs=[
        pl.BlockSpec((MBLK, D), lambda i: (i, 0)),
        pl.BlockSpec((D, D), lambda i: (0, 0)),
        pl.BlockSpec((D, D), lambda i: (0, 0)),
        pl.BlockSpec((MBLK, 1), lambda i: (i, 0)),
    ],
    out_specs=[
        pl.BlockSpec((MBLK, D), lambda i: (i, 0)),
        pl.BlockSpec((MBLK, D), lambda i: (i, 0)),
    ],
    out_shape=[
        jax.ShapeDtypeStruct((N, D), jnp.float32),
        jax.ShapeDtypeStruct((N, D), jnp.float32),
    ],
)


_sc_mesh = plsc.VectorSubcoreMesh(
    core_axis_name="c", subcore_axis_name="s", num_cores=NC, num_subcores=NS
)


@functools.partial(
    pl.kernel,
    out_type=jax.ShapeDtypeStruct((NC, N, D), jnp.float32),
    mesh=_sc_mesh,
    scratch_types=[
        pltpu.VMEM((G, CH), jnp.int32),
        pltpu.VMEM((G, CH), jnp.int32),
        pltpu.VMEM((2, CH, D), jnp.float32),
        pltpu.VMEM_SHARED((N, D), jnp.float32),
        pltpu.SemaphoreType.DMA((2,)),
    ],
)
def _sc_scatter(hw_hbm, src_hbm, dst_hbm, zeros_hbm, out_hbm,
                src_v, dst_v, rows_v, acc_sh, sems):
    cid = lax.axis_index("c")
    sid = lax.axis_index("s")
    wid = sid * NC + cid
    pltpu.sync_copy(zeros_hbm, acc_sh.at[pl.ds(sid * RSTRIDE, RWIN)])
    plsc.subcore_barrier()

    def group(g, carry):
        pltpu.sync_copy(src_hbm.at[wid, pl.ds(g * G, G)], src_v)
        pltpu.sync_copy(dst_hbm.at[wid, pl.ds(g * G, G)], dst_v)
        pltpu.async_copy(hw_hbm.at[src_v.at[0]], rows_v.at[0], sems.at[0])

        def body(j, inner):
            p = lax.rem(j, 2)
            q = lax.rem(j + 1, 2)

            @pl.when(j + 1 < G)
            def _():
                pltpu.async_copy(hw_hbm.at[src_v.at[j + 1]], rows_v.at[q], sems.at[q])

            pltpu.make_async_copy(hw_hbm.at[src_v.at[j]], rows_v.at[p], sems.at[p]).wait()
            pltpu.sync_copy(rows_v.at[p], acc_sh.at[dst_v.at[j]], add=True)
            return inner

        lax.fori_loop(0, G, body, 0)
        return carry

    lax.fori_loop(0, NGROUP, group, 0)
    plsc.subcore_barrier()
    pltpu.sync_copy(acc_sh.at[pl.ds(sid * RSTRIDE, RWIN)],
                    out_hbm.at[cid, pl.ds(sid * RSTRIDE, RWIN)])


def _fin_body(agg_ref, norm_ref, b_ref, res_ref, out_ref):
    agg = agg_ref[0] + agg_ref[1]
    out_ref[...] = jnp.maximum(agg * norm_ref[...] + b_ref[...] + res_ref[...], 0.0)


_fin_call = pl.pallas_call(
    _fin_body,
    grid=(N // MBLK,),
    in_specs=[
        pl.BlockSpec((NC, MBLK, D), lambda i: (0, i, 0)),
        pl.BlockSpec((MBLK, 1), lambda i: (i, 0)),
        pl.BlockSpec((1, D), lambda i: (0, 0)),
        pl.BlockSpec((MBLK, D), lambda i: (i, 0)),
    ],
    out_specs=pl.BlockSpec((MBLK, D), lambda i: (i, 0)),
    out_shape=jax.ShapeDtypeStruct((N, D), jnp.float32),
)


def kernel(h, edge_index, norm, W, b, W_res, b_res):
    src = edge_index[0].reshape(NW, NCHUNK, CH)
    dst = edge_index[1].reshape(NW, NCHUNK, CH)
    hw, res = _mm_call(h, W, W_res.T, norm)
    zeros = jnp.zeros((RWIN, D), jnp.float32)
    aggs = _sc_scatter(hw, src, dst, zeros)
    bias = (b + b_res).reshape(1, D)
    return _fin_call(aggs, norm, bias, res)


# speedup vs baseline: 11.2355x; 1.0549x over previous
import functools

import jax
import jax.numpy as jnp
from jax import lax
from jax.experimental import pallas as pl
from jax.experimental.pallas import tpu as pltpu
from jax.experimental.pallas import tpu_sc as plsc

N = 10000
E = 320000
D = 128

NC = 2
NS = 16
NW = NC * NS
EPW = E // NW
CH = 125
NCHUNK = EPW // CH
G = 40
NGROUP = NCHUNK // G
RSTRIDE = 624
RWIN = 640

MBLK = 1000


def _mm_body(h_ref, w_ref, norm_ref, hw_ref):
    hw_ref[...] = jnp.dot(h_ref[...], w_ref[...],
                          preferred_element_type=jnp.float32) * norm_ref[...]


_mm_call = pl.pallas_call(
    _mm_body,
    grid=(N // MBLK,),
    in_specs=[
        pl.BlockSpec((MBLK, D), lambda i: (i, 0)),
        pl.BlockSpec((D, D), lambda i: (0, 0)),
        pl.BlockSpec((MBLK, 1), lambda i: (i, 0)),
    ],
    out_specs=pl.BlockSpec((MBLK, D), lambda i: (i, 0)),
    out_shape=jax.ShapeDtypeStruct((N, D), jnp.float32),
)


_sc_mesh = plsc.VectorSubcoreMesh(
    core_axis_name="c", subcore_axis_name="s", num_cores=NC, num_subcores=NS
)


@functools.partial(
    pl.kernel,
    out_type=jax.ShapeDtypeStruct((NC, N, D), jnp.float32),
    mesh=_sc_mesh,
    scratch_types=[
        pltpu.VMEM((G, CH), jnp.int32),
        pltpu.VMEM((G, CH), jnp.int32),
        pltpu.VMEM((2, CH, D), jnp.float32),
        pltpu.VMEM_SHARED((N, D), jnp.float32),
        pltpu.SemaphoreType.DMA((2,)),
    ],
)
def _sc_scatter(hw_hbm, src_hbm, dst_hbm, zeros_hbm, out_hbm,
                src_v, dst_v, rows_v, acc_sh, sems):
    cid = lax.axis_index("c")
    sid = lax.axis_index("s")
    wid = sid * NC + cid
    pltpu.sync_copy(zeros_hbm, acc_sh.at[pl.ds(sid * RSTRIDE, RWIN)])
    plsc.subcore_barrier()

    def group(g, carry):
        pltpu.sync_copy(src_hbm.at[wid, pl.ds(g * G, G)], src_v)
        pltpu.sync_copy(dst_hbm.at[wid, pl.ds(g * G, G)], dst_v)
        pltpu.async_copy(hw_hbm.at[src_v.at[0]], rows_v.at[0], sems.at[0])

        def body(j, inner):
            p = lax.rem(j, 2)
            q = lax.rem(j + 1, 2)

            @pl.when(j + 1 < G)
            def _():
                pltpu.async_copy(hw_hbm.at[src_v.at[j + 1]], rows_v.at[q], sems.at[q])

            pltpu.make_async_copy(hw_hbm.at[src_v.at[j]], rows_v.at[p], sems.at[p]).wait()
            pltpu.sync_copy(rows_v.at[p], acc_sh.at[dst_v.at[j]], add=True)
            return inner

        lax.fori_loop(0, G, body, 0)
        return carry

    lax.fori_loop(0, NGROUP, group, 0)
    plsc.subcore_barrier()
    pltpu.sync_copy(acc_sh.at[pl.ds(sid * RSTRIDE, RWIN)],
                    out_hbm.at[cid, pl.ds(sid * RSTRIDE, RWIN)])


def _fin_body(agg_ref, norm_ref, b_ref, h_ref, wt_ref, out_ref):
    agg = agg_ref[0] + agg_ref[1]
    res = jnp.dot(h_ref[...], wt_ref[...], preferred_element_type=jnp.float32)
    out_ref[...] = jnp.maximum(agg * norm_ref[...] + b_ref[...] + res, 0.0)


_fin_call = pl.pallas_call(
    _fin_body,
    grid=(N // MBLK,),
    in_specs=[
        pl.BlockSpec((NC, MBLK, D), lambda i: (0, i, 0)),
        pl.BlockSpec((MBLK, 1), lambda i: (i, 0)),
        pl.BlockSpec((1, D), lambda i: (0, 0)),
        pl.BlockSpec((MBLK, D), lambda i: (i, 0)),
        pl.BlockSpec((D, D), lambda i: (0, 0)),
    ],
    out_specs=pl.BlockSpec((MBLK, D), lambda i: (i, 0)),
    out_shape=jax.ShapeDtypeStruct((N, D), jnp.float32),
)


def kernel(h, edge_index, norm, W, b, W_res, b_res):
    src = edge_index[0].reshape(NW, NCHUNK, CH)
    dst = edge_index[1].reshape(NW, NCHUNK, CH)
    hw = _mm_call(h, W, norm)
    zeros = jnp.zeros((RWIN, D), jnp.float32)
    aggs = _sc_scatter(hw, src, dst, zeros)
    bias = (b + b_res).reshape(1, D)
    return _fin_call(aggs, norm, bias, h, W_res.T)


# speedup vs baseline: 11.3200x; 1.0075x over previous
import functools

import jax
import jax.numpy as jnp
from jax import lax
from jax.experimental import pallas as pl
from jax.experimental.pallas import tpu as pltpu
from jax.experimental.pallas import tpu_sc as plsc

N = 10000
E = 320000
D = 128

NC = 2
NS = 16
NW = NC * NS
EPW = E // NW
CH = 125
NCHUNK = EPW // CH
G = 40
NGROUP = NCHUNK // G
RSTRIDE = 624
RWIN = 640

MBLK = 1000


def _mm_body(h_ref, w_ref, norm_ref, hw_ref):
    hw_ref[...] = jnp.dot(h_ref[...], w_ref[...],
                          preferred_element_type=jnp.float32) * norm_ref[...]


_mm_call = pl.pallas_call(
    _mm_body,
    grid=(N // MBLK,),
    in_specs=[
        pl.BlockSpec((MBLK, D), lambda i: (i, 0)),
        pl.BlockSpec((D, D), lambda i: (0, 0)),
        pl.BlockSpec((MBLK, 1), lambda i: (i, 0)),
    ],
    out_specs=pl.BlockSpec((MBLK, D), lambda i: (i, 0)),
    out_shape=jax.ShapeDtypeStruct((N, D), jnp.float32),
)


_sc_mesh = plsc.VectorSubcoreMesh(
    core_axis_name="c", subcore_axis_name="s", num_cores=NC, num_subcores=NS
)


@functools.partial(
    pl.kernel,
    out_type=jax.ShapeDtypeStruct((NC, N, D), jnp.float32),
    mesh=_sc_mesh,
    scratch_types=[
        pltpu.VMEM((G, CH), jnp.int32),
        pltpu.VMEM((G, CH), jnp.int32),
        pltpu.VMEM((2, CH, D), jnp.float32),
        pltpu.VMEM_SHARED((N, D), jnp.float32),
        pltpu.SemaphoreType.DMA((2,)),
        pltpu.SemaphoreType.DMA((2,)),
    ],
)
def _sc_scatter(hw_hbm, src_hbm, dst_hbm, zeros_hbm, out_hbm,
                src_v, dst_v, rows_v, acc_sh, gsems, ssems):
    cid = lax.axis_index("c")
    sid = lax.axis_index("s")
    wid = sid * NC + cid

    def stage_and_prime(g):
        pltpu.sync_copy(src_hbm.at[wid, pl.ds(g * G, G)], src_v)
        pltpu.sync_copy(dst_hbm.at[wid, pl.ds(g * G, G)], dst_v)
        pltpu.async_copy(hw_hbm.at[src_v.at[0]], rows_v.at[0], gsems.at[0])

    def run_group():
        def body(j, inner):
            p = lax.rem(j, 2)
            q = lax.rem(j + 1, 2)

            @pl.when(jnp.logical_and(j + 1 < G, j >= 1))
            def _():
                pltpu.make_async_copy(rows_v.at[q], acc_sh.at[dst_v.at[j - 1]],
                                      ssems.at[q]).wait()

            @pl.when(j + 1 < G)
            def _():
                pltpu.async_copy(hw_hbm.at[src_v.at[j + 1]], rows_v.at[q], gsems.at[q])

            pltpu.make_async_copy(hw_hbm.at[src_v.at[j]], rows_v.at[p], gsems.at[p]).wait()
            pltpu.async_copy(rows_v.at[p], acc_sh.at[dst_v.at[j]], ssems.at[p], add=True)
            return inner

        lax.fori_loop(0, G, body, 0)
        pltpu.make_async_copy(rows_v.at[0], acc_sh.at[dst_v.at[G - 2]], ssems.at[0]).wait()
        pltpu.make_async_copy(rows_v.at[1], acc_sh.at[dst_v.at[G - 1]], ssems.at[1]).wait()

    pltpu.sync_copy(zeros_hbm, acc_sh.at[pl.ds(sid * RSTRIDE, RWIN)])
    stage_and_prime(0)
    plsc.subcore_barrier()
    run_group()
    stage_and_prime(1)
    run_group()
    plsc.subcore_barrier()
    pltpu.sync_copy(acc_sh.at[pl.ds(sid * RSTRIDE, RWIN)],
                    out_hbm.at[cid, pl.ds(sid * RSTRIDE, RWIN)])


def _fin_body(agg_ref, norm_ref, b_ref, h_ref, wt_ref, out_ref):
    agg = agg_ref[0] + agg_ref[1]
    res = jnp.dot(h_ref[...], wt_ref[...], preferred_element_type=jnp.float32)
    out_ref[...] = jnp.maximum(agg * norm_ref[...] + b_ref[...] + res, 0.0)


_fin_call = pl.pallas_call(
    _fin_body,
    grid=(N // MBLK,),
    in_specs=[
        pl.BlockSpec((NC, MBLK, D), lambda i: (0, i, 0)),
        pl.BlockSpec((MBLK, 1), lambda i: (i, 0)),
        pl.BlockSpec((1, D), lambda i: (0, 0)),
        pl.BlockSpec((MBLK, D), lambda i: (i, 0)),
        pl.BlockSpec((D, D), lambda i: (0, 0)),
    ],
    out_specs=pl.BlockSpec((MBLK, D), lambda i: (i, 0)),
    out_shape=jax.ShapeDtypeStruct((N, D), jnp.float32),
)


def kernel(h, edge_index, norm, W, b, W_res, b_res):
    src = edge_index[0].reshape(NW, NCHUNK, CH)
    dst = edge_index[1].reshape(NW, NCHUNK, CH)
    hw = _mm_call(h, W, norm)
    zeros = jnp.zeros((RWIN, D), jnp.float32)
    aggs = _sc_scatter(hw, src, dst, zeros)
    bias = (b + b_res).reshape(1, D)
    return _fin_call(aggs, norm, bias, h, W_res.T)


# speedup vs baseline: 11.9957x; 1.0597x over previous
import functools

import jax
import jax.numpy as jnp
from jax import lax
from jax.experimental import pallas as pl
from jax.experimental.pallas import tpu as pltpu
from jax.experimental.pallas import tpu_sc as plsc

N = 10000
E = 320000
D = 128

NC = 2
NS = 16
NW = NC * NS
EPW = E // NW
CH = 125
NCHUNK = EPW // CH
G = 40
NGROUP = NCHUNK // G
RSTRIDE = 624
RWIN = 640

MBLK = 1000


def _mm_body(h_ref, w_ref, norm_ref, hw_ref):
    hw_ref[...] = jnp.dot(h_ref[...], w_ref[...],
                          preferred_element_type=jnp.float32) * norm_ref[...]


_mm_call = pl.pallas_call(
    _mm_body,
    grid=(N // MBLK,),
    in_specs=[
        pl.BlockSpec((MBLK, D), lambda i: (i, 0)),
        pl.BlockSpec((D, D), lambda i: (0, 0)),
        pl.BlockSpec((MBLK, 1), lambda i: (i, 0)),
    ],
    out_specs=pl.BlockSpec((MBLK, D), lambda i: (i, 0)),
    out_shape=jax.ShapeDtypeStruct((N, D), jnp.float32),
)


_sc_mesh = plsc.VectorSubcoreMesh(
    core_axis_name="c", subcore_axis_name="s", num_cores=NC, num_subcores=NS
)


@functools.partial(
    pl.kernel,
    out_type=jax.ShapeDtypeStruct((NC, N, D), jnp.float32),
    mesh=_sc_mesh,
    scratch_types=[
        pltpu.VMEM((G, CH), jnp.int32),
        pltpu.VMEM((G, CH), jnp.int32),
        pltpu.VMEM((2, CH, D), jnp.float32),
        pltpu.VMEM_SHARED((N, D), jnp.float32),
        pltpu.SemaphoreType.DMA((2,)),
        pltpu.SemaphoreType.DMA((2,)),
    ],
)
def _sc_scatter(hw_hbm, edges_hbm, zeros_hbm, out_hbm,
                src_v, dst_v, rows_v, acc_sh, gsems, ssems):
    cid = lax.axis_index("c")
    sid = lax.axis_index("s")
    wid = sid * NC + cid

    def stage_and_prime(g):
        pltpu.sync_copy(edges_hbm.at[0, wid, pl.ds(g * G, G)], src_v)
        pltpu.sync_copy(edges_hbm.at[1, wid, pl.ds(g * G, G)], dst_v)
        pltpu.async_copy(hw_hbm.at[src_v.at[0]], rows_v.at[0], gsems.at[0])

    def run_group():
        def body(j, inner):
            p = lax.rem(j, 2)
            q = lax.rem(j + 1, 2)

            @pl.when(jnp.logical_and(j + 1 < G, j >= 1))
            def _():
                pltpu.make_async_copy(rows_v.at[q], acc_sh.at[dst_v.at[j - 1]],
                                      ssems.at[q]).wait()

            @pl.when(j + 1 < G)
            def _():
                pltpu.async_copy(hw_hbm.at[src_v.at[j + 1]], rows_v.at[q], gsems.at[q])

            pltpu.make_async_copy(hw_hbm.at[src_v.at[j]], rows_v.at[p], gsems.at[p]).wait()
            pltpu.async_copy(rows_v.at[p], acc_sh.at[dst_v.at[j]], ssems.at[p], add=True)
            return inner

        lax.fori_loop(0, G, body, 0)
        pltpu.make_async_copy(rows_v.at[0], acc_sh.at[dst_v.at[G - 2]], ssems.at[0]).wait()
        pltpu.make_async_copy(rows_v.at[1], acc_sh.at[dst_v.at[G - 1]], ssems.at[1]).wait()

    pltpu.sync_copy(zeros_hbm, acc_sh.at[pl.ds(sid * RSTRIDE, RWIN)])
    stage_and_prime(0)
    plsc.subcore_barrier()
    run_group()
    stage_and_prime(1)
    run_group()
    plsc.subcore_barrier()
    pltpu.sync_copy(acc_sh.at[pl.ds(sid * RSTRIDE, RWIN)],
                    out_hbm.at[cid, pl.ds(sid * RSTRIDE, RWIN)])


def _fin_body(agg_ref, norm_ref, b_ref, br_ref, h_ref, wr_ref, out_ref):
    agg = agg_ref[0] + agg_ref[1]
    res = lax.dot_general(h_ref[...], wr_ref[...], (((1,), (1,)), ((), ())),
                          preferred_element_type=jnp.float32)
    bias = b_ref[...] + br_ref[...]
    out_ref[...] = jnp.maximum(agg * norm_ref[...] + bias + res, 0.0)


_fin_call = pl.pallas_call(
    _fin_body,
    grid=(N // MBLK,),
    in_specs=[
        pl.BlockSpec((NC, MBLK, D), lambda i: (0, i, 0)),
        pl.BlockSpec((MBLK, 1), lambda i: (i, 0)),
        pl.BlockSpec((1, D), lambda i: (0, 0)),
        pl.BlockSpec((1, D), lambda i: (0, 0)),
        pl.BlockSpec((MBLK, D), lambda i: (i, 0)),
        pl.BlockSpec((D, D), lambda i: (0, 0)),
    ],
    out_specs=pl.BlockSpec((MBLK, D), lambda i: (i, 0)),
    out_shape=jax.ShapeDtypeStruct((N, D), jnp.float32),
)


def kernel(h, edge_index, norm, W, b, W_res, b_res):
    edges = edge_index.reshape(2, NW, NCHUNK, CH)
    hw = _mm_call(h, W, norm)
    zeros = jnp.zeros((RWIN, D), jnp.float32)
    aggs = _sc_scatter(hw, edges, zeros)
    return _fin_call(aggs, norm, b.reshape(1, D), b_res.reshape(1, D), h, W_res)


# speedup vs baseline: 12.3958x; 1.0334x over previous
import functools

import jax
import jax.numpy as jnp
from jax import lax
from jax.experimental import pallas as pl
from jax.experimental.pallas import tpu as pltpu
from jax.experimental.pallas import tpu_sc as plsc

N = 10000
E = 320000
D = 128

NC = 2
NS = 16
NW = NC * NS
EPW = E // NW
CH = 50
NCHUNK = EPW // CH
G = 40
NGROUP = NCHUNK // G
RING = 4
RSTRIDE = 624
RWIN = 640

MBLK = 2000


def _mm_body(h_ref, w_ref, norm_ref, hw_ref):
    hw_ref[...] = jnp.dot(h_ref[...], w_ref[...],
                          preferred_element_type=jnp.float32) * norm_ref[...]


_mm_call = pl.pallas_call(
    _mm_body,
    grid=(N // MBLK,),
    in_specs=[
        pl.BlockSpec((MBLK, D), lambda i: (i, 0)),
        pl.BlockSpec((D, D), lambda i: (0, 0)),
        pl.BlockSpec((MBLK, 1), lambda i: (i, 0)),
    ],
    out_specs=pl.BlockSpec((MBLK, D), lambda i: (i, 0)),
    out_shape=jax.ShapeDtypeStruct((N, D), jnp.float32),
)


_sc_mesh = plsc.VectorSubcoreMesh(
    core_axis_name="c", subcore_axis_name="s", num_cores=NC, num_subcores=NS
)


@functools.partial(
    pl.kernel,
    out_type=jax.ShapeDtypeStruct((NC, N, D), jnp.float32),
    mesh=_sc_mesh,
    scratch_types=[
        pltpu.VMEM((G, CH), jnp.int32),
        pltpu.VMEM((G, CH), jnp.int32),
        pltpu.VMEM((RING, CH, D), jnp.float32),
        pltpu.VMEM_SHARED((N, D), jnp.float32),
        pltpu.SemaphoreType.DMA((RING,)),
        pltpu.SemaphoreType.DMA((RING,)),
    ],
)
def _sc_scatter(hw_hbm, edges_hbm, zeros_hbm, out_hbm,
                src_v, dst_v, rows_v, acc_sh, gsems, ssems):
    cid = lax.axis_index("c")
    sid = lax.axis_index("s")
    wid = sid * NC + cid

    def stage_and_prime(g):
        pltpu.sync_copy(edges_hbm.at[0, wid, pl.ds(g * G, G)], src_v)
        pltpu.sync_copy(edges_hbm.at[1, wid, pl.ds(g * G, G)], dst_v)
        for k in range(RING - 1):
            pltpu.async_copy(hw_hbm.at[src_v.at[k]], rows_v.at[k], gsems.at[k])

    pltpu.sync_copy(zeros_hbm, acc_sh.at[pl.ds(sid * RSTRIDE, RWIN)])
    stage_and_prime(0)
    plsc.subcore_barrier()

    def group(g, carry):
        def body(j, inner):
            p = lax.rem(j, RING)
            nq = lax.rem(j + RING - 1, RING)
            nxt = j + RING - 1

            @pl.when(jnp.logical_and(j >= 1, nxt < G))
            def _():
                pltpu.make_async_copy(rows_v.at[nq], acc_sh.at[dst_v.at[j - 1]],
                                      ssems.at[nq]).wait()

            @pl.when(nxt < G)
            def _():
                pltpu.async_copy(hw_hbm.at[src_v.at[nxt]], rows_v.at[nq], gsems.at[nq])

            pltpu.make_async_copy(hw_hbm.at[src_v.at[j]], rows_v.at[p], gsems.at[p]).wait()
            pltpu.async_copy(rows_v.at[p], acc_sh.at[dst_v.at[j]], ssems.at[p], add=True)
            return inner

        lax.fori_loop(0, G, body, 0)
        for c in range(G - RING, G):
            pltpu.make_async_copy(rows_v.at[c % RING], acc_sh.at[dst_v.at[c]],
                                  ssems.at[c % RING]).wait()

        @pl.when(g + 1 < NGROUP)
        def _():
            stage_and_prime(g + 1)
        return carry

    lax.fori_loop(0, NGROUP, group, 0)
    plsc.subcore_barrier()
    pltpu.sync_copy(acc_sh.at[pl.ds(sid * RSTRIDE, RWIN)],
                    out_hbm.at[cid, pl.ds(sid * RSTRIDE, RWIN)])


def _fin_body(agg_ref, norm_ref, b_ref, br_ref, h_ref, wr_ref, out_ref):
    agg = agg_ref[0] + agg_ref[1]
    res = lax.dot_general(h_ref[...], wr_ref[...], (((1,), (1,)), ((), ())),
                          preferred_element_type=jnp.float32)
    bias = b_ref[...] + br_ref[...]
    out_ref[...] = jnp.maximum(agg * norm_ref[...] + bias + res, 0.0)


_fin_call = pl.pallas_call(
    _fin_body,
    grid=(N // MBLK,),
    in_specs=[
        pl.BlockSpec((NC, MBLK, D), lambda i: (0, i, 0)),
        pl.BlockSpec((MBLK, 1), lambda i: (i, 0)),
        pl.BlockSpec((1, D), lambda i: (0, 0)),
        pl.BlockSpec((1, D), lambda i: (0, 0)),
        pl.BlockSpec((MBLK, D), lambda i: (i, 0)),
        pl.BlockSpec((D, D), lambda i: (0, 0)),
    ],
    out_specs=pl.BlockSpec((MBLK, D), lambda i: (i, 0)),
    out_shape=jax.ShapeDtypeStruct((N, D), jnp.float32),
)


def kernel(h, edge_index, norm, W, b, W_res, b_res):
    edges = edge_index.reshape(2, NW, NCHUNK, CH)
    hw = _mm_call(h, W, norm)
    zeros = jnp.zeros((RWIN, D), jnp.float32)
    aggs = _sc_scatter(hw, edges, zeros)
    return _fin_call(aggs, norm, b.reshape(1, D), b_res.reshape(1, D), h, W_res)


# speedup vs baseline: 12.4188x; 1.0019x over previous
import functools

import jax
import jax.numpy as jnp
from jax import lax
from jax.experimental import pallas as pl
from jax.experimental.pallas import tpu as pltpu
from jax.experimental.pallas import tpu_sc as plsc

N = 10000
E = 320000
D = 128

NC = 2
NS = 16
NW = NC * NS
EPW = E // NW
CH = 50
NCHUNK = EPW // CH
G = 40
NGROUP = NCHUNK // G
RING = 5
RSTRIDE = 624
RWIN = 640

MBLK = 2000


def _mm_body(h_ref, w_ref, norm_ref, hw_ref):
    hw_ref[...] = jnp.dot(h_ref[...], w_ref[...],
                          preferred_element_type=jnp.float32) * norm_ref[...]


_mm_call = pl.pallas_call(
    _mm_body,
    grid=(N // MBLK,),
    in_specs=[
        pl.BlockSpec((MBLK, D), lambda i: (i, 0)),
        pl.BlockSpec((D, D), lambda i: (0, 0)),
        pl.BlockSpec((MBLK, 1), lambda i: (i, 0)),
    ],
    out_specs=pl.BlockSpec((MBLK, D), lambda i: (i, 0)),
    out_shape=jax.ShapeDtypeStruct((N, D), jnp.float32),
)


_sc_mesh = plsc.VectorSubcoreMesh(
    core_axis_name="c", subcore_axis_name="s", num_cores=NC, num_subcores=NS
)


@functools.partial(
    pl.kernel,
    out_type=jax.ShapeDtypeStruct((NC, N, D), jnp.float32),
    mesh=_sc_mesh,
    scratch_types=[
        pltpu.VMEM((G, CH), jnp.int32),
        pltpu.VMEM((G, CH), jnp.int32),
        pltpu.VMEM((RING, CH, D), jnp.float32),
        pltpu.VMEM_SHARED((N, D), jnp.float32),
        pltpu.SemaphoreType.DMA((RING,)),
        pltpu.SemaphoreType.DMA((RING,)),
    ],
)
def _sc_scatter(hw_hbm, edges_hbm, zeros_hbm, out_hbm,
                src_v, dst_v, rows_v, acc_sh, gsems, ssems):
    cid = lax.axis_index("c")
    sid = lax.axis_index("s")
    wid = sid * NC + cid

    def stage_and_prime(g):
        pltpu.sync_copy(edges_hbm.at[0, wid, pl.ds(g * G, G)], src_v)
        pltpu.sync_copy(edges_hbm.at[1, wid, pl.ds(g * G, G)], dst_v)
        for k in range(RING - 1):
            pltpu.async_copy(hw_hbm.at[src_v.at[k]], rows_v.at[k], gsems.at[k])

    pltpu.sync_copy(zeros_hbm, acc_sh.at[pl.ds(sid * RSTRIDE, RWIN)])
    stage_and_prime(0)
    plsc.subcore_barrier()

    def group(g, carry):
        def body(j, inner):
            p = lax.rem(j, RING)
            nq = lax.rem(j + RING - 1, RING)
            nxt = j + RING - 1

            @pl.when(jnp.logical_and(j >= 1, nxt < G))
            def _():
                pltpu.make_async_copy(rows_v.at[nq], acc_sh.at[dst_v.at[j - 1]],
                                      ssems.at[nq]).wait()

            @pl.when(nxt < G)
            def _():
                pltpu.async_copy(hw_hbm.at[src_v.at[nxt]], rows_v.at[nq], gsems.at[nq])

            pltpu.make_async_copy(hw_hbm.at[src_v.at[j]], rows_v.at[p], gsems.at[p]).wait()
            pltpu.async_copy(rows_v.at[p], acc_sh.at[dst_v.at[j]], ssems.at[p], add=True)
            return inner

        lax.fori_loop(0, G, body, 0)
        for c in range(G - RING, G):
            pltpu.make_async_copy(rows_v.at[c % RING], acc_sh.at[dst_v.at[c]],
                                  ssems.at[c % RING]).wait()

        @pl.when(g + 1 < NGROUP)
        def _():
            stage_and_prime(g + 1)
        return carry

    lax.fori_loop(0, NGROUP, group, 0)
    plsc.subcore_barrier()
    pltpu.sync_copy(acc_sh.at[pl.ds(sid * RSTRIDE, RWIN)],
                    out_hbm.at[cid, pl.ds(sid * RSTRIDE, RWIN)])


def _fin_body(agg_ref, norm_ref, b_ref, br_ref, h_ref, wr_ref, out_ref):
    agg = agg_ref[0] + agg_ref[1]
    res = lax.dot_general(h_ref[...], wr_ref[...], (((1,), (1,)), ((), ())),
                          preferred_element_type=jnp.float32)
    bias = b_ref[...] + br_ref[...]
    out_ref[...] = jnp.maximum(agg * norm_ref[...] + bias + res, 0.0)


_fin_call = pl.pallas_call(
    _fin_body,
    grid=(N // MBLK,),
    in_specs=[
        pl.BlockSpec((NC, MBLK, D), lambda i: (0, i, 0)),
        pl.BlockSpec((MBLK, 1), lambda i: (i, 0)),
        pl.BlockSpec((1, D), lambda i: (0, 0)),
        pl.BlockSpec((1, D), lambda i: (0, 0)),
        pl.BlockSpec((MBLK, D), lambda i: (i, 0)),
        pl.BlockSpec((D, D), lambda i: (0, 0)),
    ],
    out_specs=pl.BlockSpec((MBLK, D), lambda i: (i, 0)),
    out_shape=jax.ShapeDtypeStruct((N, D), jnp.float32),
)


def kernel(h, edge_index, norm, W, b, W_res, b_res):
    edges = edge_index.reshape(2, NW, NCHUNK, CH)
    hw = _mm_call(h, W, norm)
    zeros = jnp.zeros((RWIN, D), jnp.float32)
    aggs = _sc_scatter(hw, edges, zeros)
    return _fin_call(aggs, norm, b.reshape(1, D), b_res.reshape(1, D), h, W_res)
